# baseline (speedup 1.0000x reference)
import jax
import jax.numpy as jnp
from jax.experimental import pallas as pl
from jax.experimental.pallas import tpu as pltpu

_BLOCK = 2000


def _hgnn_kernel(x_ref, H_ref, W1_ref, W2_ref, b1_ref, b2_ref, Wc_ref,
                 bc_ref, out_ref, xw_ref):
    f32 = jnp.float32
    i = pl.program_id(0)
    nb = pl.num_programs(0)

    xw_ref[pl.ds(i * _BLOCK, _BLOCK), :] = jnp.dot(
        x_ref[...], W1_ref[...], preferred_element_type=f32)

    @pl.when(i == nb - 1)
    def _():
        Hf = H_ref[...].astype(f32)
        ones = jnp.ones((Hf.shape[0], 1), f32)
        bdeg = jax.lax.dot_general(
            Hf, ones, (((0,), (0,)), ((), ())), preferred_element_type=f32)
        binv = jnp.where(bdeg > 0, 1.0 / bdeg, 0.0)
        deg = jnp.sum(Hf, axis=1, keepdims=True)
        dinv = jnp.where(deg > 0, 1.0 / deg, 0.0)

        m = binv * jax.lax.dot_general(
            Hf, xw_ref[...], (((0,), (0,)), ((), ())),
            preferred_element_type=f32)
        h = jax.nn.relu(
            dinv * jnp.dot(Hf, m, preferred_element_type=f32) + b1_ref[...])

        hw = jnp.dot(h, W2_ref[...], preferred_element_type=f32)
        m2 = binv * jax.lax.dot_general(
            Hf, hw, (((0,), (0,)), ((), ())), preferred_element_type=f32)
        h2 = jax.nn.relu(
            dinv * jnp.dot(Hf, m2, preferred_element_type=f32) + b2_ref[...])

        out_ref[...] = (
            jnp.dot(h2, Wc_ref[...], preferred_element_type=f32)
            + bc_ref[...])


def kernel(x, H, edge_weights, W1, b1, W2, b2, Wc, bc):
    del edge_weights
    n, d_in = x.shape
    e_h = H.shape[1]
    d_hid = W1.shape[1]
    nb = n // _BLOCK

    full = lambda *shape: pl.BlockSpec(shape, lambda i: (0,) * len(shape))

    out = pl.pallas_call(
        _hgnn_kernel,
        grid=(nb,),
        in_specs=[
            pl.BlockSpec((_BLOCK, d_in), lambda i: (i, 0)),
            full(n, e_h),
            full(d_in, d_hid),
            full(d_hid, d_hid),
            full(1, d_hid),
            full(1, d_hid),
            full(d_hid, 1),
            full(1, 1),
        ],
        out_specs=full(n, 1),
        out_shape=jax.ShapeDtypeStruct((n, 1), jnp.float32),
        scratch_shapes=[pltpu.VMEM((n, d_hid), jnp.float32)],
        compiler_params=pltpu.CompilerParams(
            dimension_semantics=("arbitrary",)),
    )(x, H, W1, W2, b1.reshape(1, d_hid), b2.reshape(1, d_hid), Wc,
      bc.reshape(1, 1))

    return out


# speedup vs baseline: 1.0688x; 1.0688x over previous
import jax
import jax.numpy as jnp
from jax.experimental import pallas as pl
from jax.experimental.pallas import tpu as pltpu


def _hgnn_kernel(x_ref, H_ref, W1_ref, W2_ref, b1_ref, b2_ref, Wc_ref,
                 bc_ref, out_ref):
    f32 = jnp.float32
    Hf = H_ref[...].astype(f32)
    ones = jnp.ones((Hf.shape[0], 1), f32)
    bdeg = jax.lax.dot_general(
        Hf, ones, (((0,), (0,)), ((), ())), preferred_element_type=f32)
    binv = jnp.where(bdeg > 0, 1.0 / bdeg, 0.0)
    deg = jnp.sum(Hf, axis=1, keepdims=True)
    dinv = jnp.where(deg > 0, 1.0 / deg, 0.0)

    xw = jnp.dot(x_ref[...], W1_ref[...], preferred_element_type=f32)
    m = binv * jax.lax.dot_general(
        Hf, xw, (((0,), (0,)), ((), ())), preferred_element_type=f32)
    h = jax.nn.relu(
        dinv * jnp.dot(Hf, m, preferred_element_type=f32) + b1_ref[...])

    hw = jnp.dot(h, W2_ref[...], preferred_element_type=f32)
    m2 = binv * jax.lax.dot_general(
        Hf, hw, (((0,), (0,)), ((), ())), preferred_element_type=f32)
    h2 = jax.nn.relu(
        dinv * jnp.dot(Hf, m2, preferred_element_type=f32) + b2_ref[...])

    out_ref[...] = (
        jnp.dot(h2, Wc_ref[...], preferred_element_type=f32) + bc_ref[...])


def kernel(x, H, edge_weights, W1, b1, W2, b2, Wc, bc):
    del edge_weights
    n, d_in = x.shape
    d_hid = W1.shape[1]

    out = pl.pallas_call(
        _hgnn_kernel,
        out_shape=jax.ShapeDtypeStruct((n, 1), jnp.float32),
    )(x, H, W1, W2, b1.reshape(1, d_hid), b2.reshape(1, d_hid), Wc,
      bc.reshape(1, 1))

    return out
